# Initial kernel scaffold; baseline (speedup 1.0000x reference)
import jax
import jax.numpy as jnp
from jax.experimental import pallas as pl


def kernel(x):
    raise NotImplementedError("write your pallas kernel here")


# speedup vs baseline: 2.7185x; 2.7185x over previous
import jax
import jax.numpy as jnp
from jax.experimental import pallas as pl

_MASK_RATIO = 0.25


def _copy_body(x_ref, o_ref):
    o_ref[...] = x_ref[...]


def _mask_body(o_ref):
    b, s = o_ref.shape
    nv = s - int(s * _MASK_RATIO)
    col = jax.lax.broadcasted_iota(jnp.int32, (b, s), 1)
    o_ref[...] = col >= nv


def kernel(x):
    b, s, f = x.shape
    num_mask = int(s * _MASK_RATIO)
    nv = s - num_mask

    bs = 512
    visible = pl.pallas_call(
        _copy_body,
        grid=(b, nv // bs),
        in_specs=[pl.BlockSpec((1, bs, f), lambda i, j: (i, j, 0))],
        out_specs=pl.BlockSpec((1, bs, f), lambda i, j: (i, j, 0)),
        out_shape=jax.ShapeDtypeStruct((b, nv, f), x.dtype),
    )(x)

    nvb = nv // bs
    masked = pl.pallas_call(
        _copy_body,
        grid=(b, num_mask // bs),
        in_specs=[pl.BlockSpec((1, bs, f), lambda i, j, nvb=nvb: (i, j + nvb, 0))],
        out_specs=pl.BlockSpec((1, bs, f), lambda i, j: (i, j, 0)),
        out_shape=jax.ShapeDtypeStruct((b, num_mask, f), x.dtype),
    )(x)

    mask = pl.pallas_call(
        _mask_body,
        out_shape=jax.ShapeDtypeStruct((b, s), jnp.bool_),
    )()

    return visible, masked, mask


# speedup vs baseline: 2.8066x; 1.0324x over previous
import functools

import jax
import jax.numpy as jnp
from jax.experimental import pallas as pl

_MASK_RATIO = 0.25


def _body(x_ref, vis_ref, msk_ref, mask_ref, *, nvb, nv):
    i = pl.program_id(0)
    j = pl.program_id(1)

    @pl.when(jnp.logical_and(i == 0, j == 0))
    def _():
        b, s = mask_ref.shape
        col = jax.lax.broadcasted_iota(jnp.int32, (b, s), 1)
        mask_ref[...] = col >= nv

    @pl.when(j < nvb)
    def _():
        vis_ref[...] = x_ref[...]

    @pl.when(j >= nvb)
    def _():
        msk_ref[...] = x_ref[...]


def kernel(x):
    b, s, f = x.shape
    num_mask = int(s * _MASK_RATIO)
    nv = s - num_mask

    bs = 512
    nvb = nv // bs

    visible, masked, mask = pl.pallas_call(
        functools.partial(_body, nvb=nvb, nv=nv),
        grid=(b, s // bs),
        in_specs=[pl.BlockSpec((1, bs, f), lambda i, j: (i, j, 0))],
        out_specs=[
            pl.BlockSpec((1, bs, f), lambda i, j: (i, jnp.minimum(j, nvb - 1), 0)),
            pl.BlockSpec((1, bs, f), lambda i, j: (i, jnp.maximum(j - nvb, 0), 0)),
            pl.BlockSpec((b, s), lambda i, j: (0, 0)),
        ],
        out_shape=[
            jax.ShapeDtypeStruct((b, nv, f), x.dtype),
            jax.ShapeDtypeStruct((b, num_mask, f), x.dtype),
            jax.ShapeDtypeStruct((b, s), jnp.bool_),
        ],
    )(x)

    return visible, masked, mask


# speedup vs baseline: 2.8207x; 1.0050x over previous
import functools

import jax
import jax.numpy as jnp
from jax.experimental import pallas as pl

_MASK_RATIO = 0.25


def _body(x_ref, vis_ref, msk_ref, mask_ref, *, nvb, nv):
    i = pl.program_id(0)
    j = pl.program_id(1)

    @pl.when(jnp.logical_and(i == 0, j == 0))
    def _():
        b, s = mask_ref.shape
        col = jax.lax.broadcasted_iota(jnp.int32, (b, s), 1)
        mask_ref[...] = col >= nv

    @pl.when(j < nvb)
    def _():
        vis_ref[...] = x_ref[...]

    @pl.when(j >= nvb)
    def _():
        msk_ref[...] = x_ref[...]


def kernel(x):
    b, s, f = x.shape
    num_mask = int(s * _MASK_RATIO)
    nv = s - num_mask

    bs = 1024
    nvb = nv // bs

    visible, masked, mask = pl.pallas_call(
        functools.partial(_body, nvb=nvb, nv=nv),
        grid=(b, s // bs),
        in_specs=[pl.BlockSpec((1, bs, f), lambda i, j: (i, j, 0))],
        out_specs=[
            pl.BlockSpec((1, bs, f), lambda i, j: (i, jnp.minimum(j, nvb - 1), 0)),
            pl.BlockSpec((1, bs, f), lambda i, j: (i, jnp.maximum(j - nvb, 0), 0)),
            pl.BlockSpec((b, s), lambda i, j: (0, 0)),
        ],
        out_shape=[
            jax.ShapeDtypeStruct((b, nv, f), x.dtype),
            jax.ShapeDtypeStruct((b, num_mask, f), x.dtype),
            jax.ShapeDtypeStruct((b, s), jnp.bool_),
        ],
    )(x)

    return visible, masked, mask
